# Initial kernel scaffold; baseline (speedup 1.0000x reference)
import jax
import jax.numpy as jnp
from jax.experimental import pallas as pl


def kernel(x, edge_index, edge_weight, params):
    raise NotImplementedError("write your pallas kernel here")


# speedup vs baseline: 21.5801x; 21.5801x over previous
import functools

import jax
import jax.numpy as jnp
from jax.experimental import pallas as pl
from jax.experimental.pallas import tpu as pltpu

_K_CHEB = 3


def _round_up(v, m):
    return (v + m - 1) // m * m


def _mm_body(a_ref, b_ref, o_ref, acc_ref):
    k = pl.program_id(2)

    @pl.when(k == 0)
    def _init():
        acc_ref[...] = jnp.zeros_like(acc_ref)

    acc_ref[...] += jnp.dot(a_ref[...], b_ref[...],
                            preferred_element_type=jnp.float32,
                            precision=jax.lax.Precision.HIGHEST)

    @pl.when(k == pl.num_programs(2) - 1)
    def _flush():
        o_ref[...] = acc_ref[...]


def _matmul(a, b, bm=1024, bn=1280, bk=512):
    m, k = a.shape
    _, n = b.shape
    assert m % bm == 0 and n % bn == 0 and k % bk == 0, (a.shape, b.shape, bm, bn, bk)
    return pl.pallas_call(
        _mm_body,
        grid=(m // bm, n // bn, k // bk),
        in_specs=[
            pl.BlockSpec((bm, bk), lambda i, j, kk: (i, kk)),
            pl.BlockSpec((bk, bn), lambda i, j, kk: (kk, j)),
        ],
        out_specs=pl.BlockSpec((bm, bn), lambda i, j, kk: (i, j)),
        out_shape=jax.ShapeDtypeStruct((m, n), jnp.float32),
        scratch_shapes=[pltpu.VMEM((bm, bn), jnp.float32)],
        compiler_params=pltpu.CompilerParams(
            dimension_semantics=("parallel", "parallel", "arbitrary")),
    )(a, b)


def _conv2d(x, w, b):
    y = jax.lax.conv_general_dilated(
        x, w, (1, 1), 'VALID', dimension_numbers=('NCHW', 'OIHW', 'NCHW'))
    return y + b[None, :, None, None]


def _temporal_conv(X, p):
    Xp = jnp.transpose(X, (0, 3, 2, 1))
    P = _conv2d(Xp, p['w1'], p['b1'])
    Q = jax.nn.sigmoid(_conv2d(Xp, p['w2'], p['b2']))
    H = jax.nn.relu(P * Q + _conv2d(Xp, p['w3'], p['b3']))
    return jnp.transpose(H, (0, 3, 2, 1))


def _build_dense_lap(edge_index, edge_weight, n, npad):
    row, col = edge_index[0], edge_index[1]
    w = jnp.where(row == col, 0.0, edge_weight)
    deg = jnp.zeros((n,), jnp.float32).at[row].add(w)
    deg_safe = jnp.where(deg > 0, deg, 1.0)
    dis = jnp.where(deg > 0, 1.0 / jnp.sqrt(deg_safe), 0.0)
    norm = -dis[row] * w * dis[col]
    A = jnp.zeros((npad, npad), jnp.float32).at[row, col].add(norm)
    return A


def _prop(A, z, npad):
    M, n, C = z.shape
    F = M * C
    Z = z.transpose(1, 0, 2).reshape(n, F)
    Z = jnp.pad(Z, ((0, npad - n), (0, 0)))
    bn = F if F <= 1536 else F // 2
    Y = _matmul(A, Z, bn=bn)[:n]
    return Y.reshape(n, M, C).transpose(1, 0, 2)


def _cheb_conv(A, z, Ws, bias, npad):
    Tx0 = z
    out = jnp.einsum('mnc,dc->mnd', Tx0, Ws[0])
    Tx1 = _prop(A, z, npad)
    out = out + jnp.einsum('mnc,dc->mnd', Tx1, Ws[1])
    for k in range(2, _K_CHEB):
        Tx2 = 2.0 * _prop(A, Tx1, npad) - Tx0
        out = out + jnp.einsum('mnc,dc->mnd', Tx2, Ws[k])
        Tx0, Tx1 = Tx1, Tx2
    return out + bias


def _batch_norm(x, gamma, beta, eps=1e-5):
    mean = jnp.mean(x, axis=(0, 2, 3), keepdims=True)
    var = jnp.var(x, axis=(0, 2, 3), keepdims=True)
    xh = (x - mean) / jnp.sqrt(var + eps)
    return xh * gamma[None, :, None, None] + beta[None, :, None, None]


def _stconv(X, A, p, npad):
    T0 = _temporal_conv(X, p['tc1'])
    B, T, n, C = T0.shape
    z = T0.reshape(B * T, n, C)
    z = _cheb_conv(A, z, p['cheb_w'], p['cheb_b'], npad)
    T1 = jax.nn.relu(z.reshape(B, T, n, C))
    T1 = _temporal_conv(T1, p['tc2'])
    Tp = jnp.transpose(T1, (0, 2, 1, 3))
    Tp = _batch_norm(Tp, p['bn_g'], p['bn_b'])
    return jnp.transpose(Tp, (0, 2, 1, 3))


def _layer_norm2(x, w, b, eps=1e-5):
    mean = jnp.mean(x, axis=(-2, -1), keepdims=True)
    var = jnp.var(x, axis=(-2, -1), keepdims=True)
    return (x - mean) / jnp.sqrt(var + eps) * w + b


def _output_layer(x, p):
    x = _conv2d(x, p['tc1_w'], p['tc1_b'])
    x = jnp.transpose(x, (0, 2, 3, 1))
    x = _layer_norm2(x, p['ln_w'], p['ln_b'])
    x = jnp.transpose(x, (0, 3, 1, 2))
    x = _conv2d(x, p['tc2_w'], p['tc2_b'])
    x = _conv2d(x, p['fc_w'], p['fc_b'])
    return x


def kernel(x, edge_index, edge_weight, params):
    n = x.shape[2]
    npad = _round_up(n, 1024)
    A = _build_dense_lap(edge_index, edge_weight, n, npad)
    for l in range(2):
        x = _stconv(x, A, params['layer%d' % l], npad)
    x = jnp.transpose(x, (0, 3, 1, 2))
    return _output_layer(x, params['out'])


# speedup vs baseline: 28.3539x; 1.3139x over previous
import functools

import jax
import jax.numpy as jnp
from jax.experimental import pallas as pl
from jax.experimental.pallas import tpu as pltpu

_K_CHEB = 3


def _round_up(v, m):
    return (v + m - 1) // m * m


def _mm_body(a_hi_ref, a_lo_ref, b_ref, o_ref, acc_ref):
    k = pl.program_id(2)

    @pl.when(k == 0)
    def _init():
        acc_ref[...] = jnp.zeros_like(acc_ref)

    b = b_ref[...]
    b_hi = b.astype(jnp.bfloat16)
    b_lo = (b - b_hi.astype(jnp.float32)).astype(jnp.bfloat16)
    a_hi = a_hi_ref[...]
    acc_ref[...] += (
        jnp.dot(a_hi, b_hi, preferred_element_type=jnp.float32)
        + jnp.dot(a_lo_ref[...], b_hi, preferred_element_type=jnp.float32)
        + jnp.dot(a_hi, b_lo, preferred_element_type=jnp.float32))

    @pl.when(k == pl.num_programs(2) - 1)
    def _flush():
        o_ref[...] = acc_ref[...]


def _matmul(a_hi, a_lo, b, bm=1024, bn=1280, bk=512):
    m, k = a_hi.shape
    _, n = b.shape
    assert m % bm == 0 and n % bn == 0 and k % bk == 0, (a_hi.shape, b.shape, bm, bn, bk)
    return pl.pallas_call(
        _mm_body,
        grid=(m // bm, n // bn, k // bk),
        in_specs=[
            pl.BlockSpec((bm, bk), lambda i, j, kk: (i, kk)),
            pl.BlockSpec((bm, bk), lambda i, j, kk: (i, kk)),
            pl.BlockSpec((bk, bn), lambda i, j, kk: (kk, j)),
        ],
        out_specs=pl.BlockSpec((bm, bn), lambda i, j, kk: (i, j)),
        out_shape=jax.ShapeDtypeStruct((m, n), jnp.float32),
        scratch_shapes=[pltpu.VMEM((bm, bn), jnp.float32)],
        compiler_params=pltpu.CompilerParams(
            dimension_semantics=("parallel", "parallel", "arbitrary")),
    )(a_hi, a_lo, b)


def _conv2d(x, w, b):
    y = jax.lax.conv_general_dilated(
        x, w, (1, 1), 'VALID', dimension_numbers=('NCHW', 'OIHW', 'NCHW'))
    return y + b[None, :, None, None]


def _temporal_conv(X, p):
    Xp = jnp.transpose(X, (0, 3, 2, 1))
    P = _conv2d(Xp, p['w1'], p['b1'])
    Q = jax.nn.sigmoid(_conv2d(Xp, p['w2'], p['b2']))
    H = jax.nn.relu(P * Q + _conv2d(Xp, p['w3'], p['b3']))
    return jnp.transpose(H, (0, 3, 2, 1))


def _build_dense_lap(edge_index, edge_weight, n, npad):
    row, col = edge_index[0], edge_index[1]
    w = jnp.where(row == col, 0.0, edge_weight)
    deg = jnp.zeros((n,), jnp.float32).at[row].add(w)
    deg_safe = jnp.where(deg > 0, deg, 1.0)
    dis = jnp.where(deg > 0, 1.0 / jnp.sqrt(deg_safe), 0.0)
    norm = -dis[row] * w * dis[col]
    A = jnp.zeros((npad, npad), jnp.float32).at[row, col].add(norm)
    return A


def _prop(A2, z, npad):
    M, n, C = z.shape
    F = M * C
    Z = z.transpose(1, 0, 2).reshape(n, F)
    Z = jnp.pad(Z, ((0, npad - n), (0, 0)))
    bn = F if F <= 1536 else F // 2
    Y = _matmul(A2[0], A2[1], Z, bn=bn)[:n]
    return Y.reshape(n, M, C).transpose(1, 0, 2)


def _cheb_conv(A2, z, Ws, bias, npad):
    Tx0 = z
    out = jnp.einsum('mnc,dc->mnd', Tx0, Ws[0])
    Tx1 = _prop(A2, z, npad)
    out = out + jnp.einsum('mnc,dc->mnd', Tx1, Ws[1])
    for k in range(2, _K_CHEB):
        Tx2 = 2.0 * _prop(A2, Tx1, npad) - Tx0
        out = out + jnp.einsum('mnc,dc->mnd', Tx2, Ws[k])
        Tx0, Tx1 = Tx1, Tx2
    return out + bias


def _batch_norm(x, gamma, beta, eps=1e-5):
    mean = jnp.mean(x, axis=(0, 2, 3), keepdims=True)
    var = jnp.var(x, axis=(0, 2, 3), keepdims=True)
    xh = (x - mean) / jnp.sqrt(var + eps)
    return xh * gamma[None, :, None, None] + beta[None, :, None, None]


def _stconv(X, A2, p, npad):
    T0 = _temporal_conv(X, p['tc1'])
    B, T, n, C = T0.shape
    z = T0.reshape(B * T, n, C)
    z = _cheb_conv(A2, z, p['cheb_w'], p['cheb_b'], npad)
    T1 = jax.nn.relu(z.reshape(B, T, n, C))
    T1 = _temporal_conv(T1, p['tc2'])
    Tp = jnp.transpose(T1, (0, 2, 1, 3))
    Tp = _batch_norm(Tp, p['bn_g'], p['bn_b'])
    return jnp.transpose(Tp, (0, 2, 1, 3))


def _layer_norm2(x, w, b, eps=1e-5):
    mean = jnp.mean(x, axis=(-2, -1), keepdims=True)
    var = jnp.var(x, axis=(-2, -1), keepdims=True)
    return (x - mean) / jnp.sqrt(var + eps) * w + b


def _output_layer(x, p):
    x = _conv2d(x, p['tc1_w'], p['tc1_b'])
    x = jnp.transpose(x, (0, 2, 3, 1))
    x = _layer_norm2(x, p['ln_w'], p['ln_b'])
    x = jnp.transpose(x, (0, 3, 1, 2))
    x = _conv2d(x, p['tc2_w'], p['tc2_b'])
    x = _conv2d(x, p['fc_w'], p['fc_b'])
    return x


def kernel(x, edge_index, edge_weight, params):
    n = x.shape[2]
    npad = _round_up(n, 1024)
    A = _build_dense_lap(edge_index, edge_weight, n, npad)
    A_hi = A.astype(jnp.bfloat16)
    A_lo = (A - A_hi.astype(jnp.float32)).astype(jnp.bfloat16)
    A2 = (A_hi, A_lo)
    for l in range(2):
        x = _stconv(x, A2, params['layer%d' % l], npad)
    x = jnp.transpose(x, (0, 3, 1, 2))
    return _output_layer(x, params['out'])
